# baseline (speedup 1.0000x reference)
import dataclasses

import jax
import jax.numpy as jnp
from jax import lax
from jax.experimental import pallas as pl
from jax.experimental.pallas import tpu as pltpu
from jax.experimental.pallas import tpu_sc as plsc

VOCAB = 1000
EMBED = 128
MEM = 1000000
NGRAM = 4
B, W, O = 1024, 200, 8

PRIMES = (131, 4062, 125923, 3903614)

NC, NS, LANES = 2, 16, 16
NW = NC * NS
ROWS_PER_W = B // NW
N_PER_W = ROWS_PER_W * W
CH = 128
NCH = N_PER_W // CH
WCHUNKS = (W + LANES - 1) // LANES
SEQ = O + W
NBUF = 6


def _maybe_when(cond, fn):
  if isinstance(cond, bool):
    if cond:
      fn()
  else:
    pl.when(cond)(fn)


def _engram_body(seq_hbm, table_hbm, gate_hbm, out_hbm,
                 seq_v, idx_v, bufs, gate_v, gsems, osems):
  wid = lax.axis_index("s") * NC + lax.axis_index("c")
  base = wid * N_PER_W

  pltpu.sync_copy(seq_hbm.at[pl.ds(wid * ROWS_PER_W, ROWS_PER_W)], seq_v)
  pltpu.sync_copy(gate_hbm, gate_v)

  for c8 in range(EMBED // LANES):
    g = gate_v[pl.ds(c8 * LANES, LANES)]
    gate_v[pl.ds(c8 * LANES, LANES)] = 1.0 / (1.0 + jnp.exp(-g))

  lane = lax.broadcasted_iota(jnp.int32, (LANES,), 0)

  def hash_row(r):
    rvec = jnp.full((LANES,), r, jnp.int32)

    @pl.loop(0, WCHUNKS)
    def _(c):
      pos = c * LANES + lane + O
      h = jnp.zeros((LANES,), jnp.uint32)
      for i in range(NGRAM):
        sval = plsc.load_gather(seq_v, [rvec, jnp.clip(pos - i, 0, SEQ - 1)])
        h = h + sval.astype(jnp.uint32) * jnp.uint32(PRIMES[i])
      look = (h % jnp.uint32(MEM)).astype(jnp.int32)
      off = r * W + c * LANES
      idx_v[off // CH, pl.ds(lax.rem(off, CH), LANES)] = look

  HEAD_ROWS = 2

  def gather_start(j, buf, gsem):
    pltpu.async_copy(table_hbm.at[idx_v.at[j]], buf, gsem)

  def step(j, buf, gsem, osem, buf2, gsem2, osem2):
    def wait_out_prev():
      pltpu.make_async_copy(
          buf2, out_hbm.at[pl.ds(base + (j + 2 - NBUF) * CH, CH)],
          osem2).wait()
    _maybe_when(j >= NBUF - 2, wait_out_prev)

    def start_jp2():
      gather_start(j + 2, buf2, gsem2)
    _maybe_when(j + 2 < NCH, start_jp2)

    def hash_next():
      hash_row(j + HEAD_ROWS)
    _maybe_when(j + HEAD_ROWS < ROWS_PER_W, hash_next)

    pltpu.make_async_copy(table_hbm.at[idx_v.at[j]], buf, gsem).wait()
    for c8 in range(EMBED // LANES):
      sl = pl.ds(c8 * LANES, LANES)
      g = gate_v[sl]

      @pl.loop(0, CH, step=8)
      def _(rr):
        for u in range(8):
          buf[rr + u, sl] = buf[rr + u, sl] * g

    pltpu.async_copy(buf, out_hbm.at[pl.ds(base + j * CH, CH)], osem)

  hash_row(0)
  gather_start(0, bufs[0], gsems[0])
  hash_row(1)
  gather_start(1, bufs[1], gsems[1])

  @pl.loop(0, (NCH - 2) // NBUF)
  def _(k):
    j0 = NBUF * k
    for u in range(NBUF):
      b, b2 = u % NBUF, (u + 2) % NBUF
      step(j0 + u, bufs[b], gsems[b], osems[b], bufs[b2], gsems[b2], osems[b2])

  for j in range(NCH - 2, NCH):
    b, b2 = j % NBUF, (j + 2) % NBUF
    step(j, bufs[b], gsems[b], osems[b], bufs[b2], gsems[b2], osems[b2])

  for j in range(NCH - NBUF + 2, NCH):
    b = j % NBUF
    pltpu.make_async_copy(
        bufs[b], out_hbm.at[pl.ds(base + j * CH, CH)], osems[b]).wait()


def _body(seq_hbm, table_hbm, gate_hbm, out_hbm,
          seq_v, idx_v,
          buf0, buf1, buf2, buf3, buf4, buf5, gate_v,
          gsem0, gsem1, gsem2, gsem3, gsem4, gsem5,
          osem0, osem1, osem2, osem3, osem4, osem5):
  _engram_body(seq_hbm, table_hbm, gate_hbm, out_hbm,
               seq_v, idx_v,
               (buf0, buf1, buf2, buf3, buf4, buf5), gate_v,
               (gsem0, gsem1, gsem2, gsem3, gsem4, gsem5),
               (osem0, osem1, osem2, osem3, osem4, osem5))


@jax.jit
def kernel(current_ids, prev_ids_overlap, memory_table, gate_logit):
  seq = jnp.concatenate([prev_ids_overlap, current_ids], axis=1)
  mesh = plsc.VectorSubcoreMesh(core_axis_name="c", subcore_axis_name="s",
                                num_cores=NC, num_subcores=NS)
  cp = pltpu.CompilerParams()
  if "needs_layout_passes" in pltpu.CompilerParams.__dataclass_fields__:
    cp = dataclasses.replace(cp, needs_layout_passes=False)
  run = pl.kernel(
      _body,
      out_type=jax.ShapeDtypeStruct((B * W, EMBED), jnp.float32),
      mesh=mesh,
      scratch_types=[
          pltpu.VMEM((ROWS_PER_W, SEQ), jnp.int32),
          pltpu.VMEM((NCH + 1, CH), jnp.int32),
          pltpu.VMEM((CH, EMBED), jnp.float32),
          pltpu.VMEM((CH, EMBED), jnp.float32),
          pltpu.VMEM((CH, EMBED), jnp.float32),
          pltpu.VMEM((CH, EMBED), jnp.float32),
          pltpu.VMEM((CH, EMBED), jnp.float32),
          pltpu.VMEM((CH, EMBED), jnp.float32),
          pltpu.VMEM((EMBED,), jnp.float32),
      ] + [pltpu.SemaphoreType.DMA] * (2 * NBUF),
      compiler_params=cp,
  )
  out = run(seq, memory_table, gate_logit)
  return out.reshape(B, W, EMBED)


# speedup vs baseline: 1.0080x; 1.0080x over previous
import dataclasses

import jax
import jax.numpy as jnp
from jax import lax
from jax.experimental import pallas as pl
from jax.experimental.pallas import tpu as pltpu
from jax.experimental.pallas import tpu_sc as plsc

VOCAB = 1000
EMBED = 128
MEM = 1000000
NGRAM = 4
B, W, O = 1024, 200, 8

PRIMES = (131, 4062, 125923, 3903614)

NC, NS, LANES = 2, 16, 16
NW = NC * NS
ROWS_PER_W = B // NW
N_PER_W = ROWS_PER_W * W
CH = 128
NCH = N_PER_W // CH
WCHUNKS = (W + LANES - 1) // LANES
SEQ = O + W
NBUF = 4


def _maybe_when(cond, fn):
  if isinstance(cond, bool):
    if cond:
      fn()
  else:
    pl.when(cond)(fn)


def _engram_body(seq_hbm, table_hbm, gate_hbm, out_hbm,
                 seq_v, idx_v, bufs, gate_v, gsems, osems):
  wid = lax.axis_index("s") * NC + lax.axis_index("c")
  base = wid * N_PER_W

  pltpu.sync_copy(seq_hbm.at[pl.ds(wid * ROWS_PER_W, ROWS_PER_W)], seq_v)
  pltpu.sync_copy(gate_hbm, gate_v)

  for c8 in range(EMBED // LANES):
    g = gate_v[pl.ds(c8 * LANES, LANES)]
    gate_v[pl.ds(c8 * LANES, LANES)] = 1.0 / (1.0 + jnp.exp(-g))

  lane = lax.broadcasted_iota(jnp.int32, (LANES,), 0)

  def hash_row(r):
    rvec = jnp.full((LANES,), r, jnp.int32)

    @pl.loop(0, WCHUNKS)
    def _(c):
      pos = c * LANES + lane + O
      h = jnp.zeros((LANES,), jnp.uint32)
      for i in range(NGRAM):
        sval = plsc.load_gather(seq_v, [rvec, jnp.clip(pos - i, 0, SEQ - 1)])
        h = h + sval.astype(jnp.uint32) * jnp.uint32(PRIMES[i])
      look = (h % jnp.uint32(MEM)).astype(jnp.int32)
      off = r * W + c * LANES
      idx_v[off // CH, pl.ds(lax.rem(off, CH), LANES)] = look

  HEAD_ROWS = 2

  def gather_start(j, buf, gsem):
    pltpu.async_copy(table_hbm.at[idx_v.at[j]], buf, gsem)

  def step(j, buf, gsem, osem, buf2, gsem2, osem2):
    def wait_out_jm2():
      pltpu.make_async_copy(
          buf2, out_hbm.at[pl.ds(base + (j - 2) * CH, CH)], osem2).wait()
    _maybe_when(j >= 2 if isinstance(j, int) else j >= 2, wait_out_jm2)

    def start_jp2():
      gather_start(j + 2, buf2, gsem2)
    _maybe_when(j + 2 < NCH, start_jp2)

    def hash_next():
      hash_row(j + HEAD_ROWS)
    _maybe_when(j + HEAD_ROWS < ROWS_PER_W, hash_next)

    pltpu.make_async_copy(table_hbm.at[idx_v.at[j]], buf, gsem).wait()
    for c8 in range(EMBED // LANES):
      sl = pl.ds(c8 * LANES, LANES)
      g = gate_v[sl]

      @pl.loop(0, CH, step=8)
      def _(rr):
        for u in range(8):
          buf[rr + u, sl] = buf[rr + u, sl] * g

    pltpu.async_copy(buf, out_hbm.at[pl.ds(base + j * CH, CH)], osem)

  hash_row(0)
  gather_start(0, bufs[0], gsems[0])
  hash_row(1)
  gather_start(1, bufs[1], gsems[1])

  @pl.loop(0, (NCH - 2) // NBUF)
  def _(k):
    j0 = NBUF * k
    for u in range(NBUF):
      b, b2 = u % NBUF, (u + 2) % NBUF
      step(j0 + u, bufs[b], gsems[b], osems[b], bufs[b2], gsems[b2], osems[b2])

  for j in range(NCH - 2, NCH):
    b, b2 = j % NBUF, (j + 2) % NBUF
    step(j, bufs[b], gsems[b], osems[b], bufs[b2], gsems[b2], osems[b2])

  for j in range(NCH - 2, NCH):
    b = j % NBUF
    pltpu.make_async_copy(
        bufs[b], out_hbm.at[pl.ds(base + j * CH, CH)], osems[b]).wait()


def _body(seq_hbm, table_hbm, gate_hbm, out_hbm,
          seq_v, idx_v,
          buf0, buf1, buf2, buf3, gate_v,
          gsem0, gsem1, gsem2, gsem3, osem0, osem1, osem2, osem3):
  _engram_body(seq_hbm, table_hbm, gate_hbm, out_hbm,
               seq_v, idx_v,
               (buf0, buf1, buf2, buf3), gate_v,
               (gsem0, gsem1, gsem2, gsem3),
               (osem0, osem1, osem2, osem3))


@jax.jit
def kernel(current_ids, prev_ids_overlap, memory_table, gate_logit):
  seq = jnp.concatenate([prev_ids_overlap, current_ids], axis=1)
  mesh = plsc.VectorSubcoreMesh(core_axis_name="c", subcore_axis_name="s",
                                num_cores=NC, num_subcores=NS)
  cp = pltpu.CompilerParams()
  if "needs_layout_passes" in pltpu.CompilerParams.__dataclass_fields__:
    cp = dataclasses.replace(cp, needs_layout_passes=False)
  run = pl.kernel(
      _body,
      out_type=jax.ShapeDtypeStruct((B * W, EMBED), jnp.float32),
      mesh=mesh,
      scratch_types=[
          pltpu.VMEM((ROWS_PER_W, SEQ), jnp.int32),
          pltpu.VMEM((NCH + 1, CH), jnp.int32),
          pltpu.VMEM((CH, EMBED), jnp.float32),
          pltpu.VMEM((CH, EMBED), jnp.float32),
          pltpu.VMEM((CH, EMBED), jnp.float32),
          pltpu.VMEM((CH, EMBED), jnp.float32),
          pltpu.VMEM((EMBED,), jnp.float32),
          pltpu.SemaphoreType.DMA,
          pltpu.SemaphoreType.DMA,
          pltpu.SemaphoreType.DMA,
          pltpu.SemaphoreType.DMA,
          pltpu.SemaphoreType.DMA,
          pltpu.SemaphoreType.DMA,
          pltpu.SemaphoreType.DMA,
          pltpu.SemaphoreType.DMA,
      ],
      compiler_params=cp,
  )
  out = run(seq, memory_table, gate_logit)
  return out.reshape(B, W, EMBED)
